# baseline (speedup 1.0000x reference)
import functools

import jax
import jax.numpy as jnp
from jax import lax
from jax.experimental import pallas as pl
from jax.experimental.pallas import tpu as pltpu
from jax.experimental.pallas import tpu_sc as plsc

N_NODES = 10000
D = 256
EPS = 0.5
NC = 2
NS = 16
NW = NC * NS
MASK_HI = -65536
MASK_LO = 0xFFFF


def _mm_bias(x, W, b, block):
    N, K = x.shape
    M = W.shape[1]
    return pl.pallas_call(
        lambda xr, wr, br, outr: outr.__setitem__(
            ..., jnp.dot(xr[...], wr[...], preferred_element_type=jnp.float32)
            + 0.5 * br[...]),
        grid=(N // block,),
        in_specs=[
            pl.BlockSpec((block, K), lambda i: (i, 0)),
            pl.BlockSpec((K, M), lambda i: (0, 0)),
            pl.BlockSpec((1, M), lambda i: (0, 0)),
        ],
        out_specs=pl.BlockSpec((block, M), lambda i: (i, 0)),
        out_shape=jax.ShapeDtypeStruct((N, M), jnp.float32),
    )(x, W, b)


def _mid_chain(t, W1, b1, W2, b2, block):
    N, K = t.shape
    M = W2.shape[1]

    def body(tr, w1r, b1r, w2r, b2r, outr):
        h = jnp.maximum(tr[...], 0.0)
        h = jnp.dot(h, w1r[...], preferred_element_type=jnp.float32) + b1r[...]
        h = jnp.maximum(EPS * h, 0.0)
        outr[...] = (jnp.dot(h, w2r[...], preferred_element_type=jnp.float32)
                     + 0.5 * b2r[...]).astype(jnp.bfloat16)

    return pl.pallas_call(
        body,
        grid=(N // block,),
        in_specs=[
            pl.BlockSpec((block, K), lambda i: (i, 0)),
            pl.BlockSpec((K, W1.shape[1]), lambda i: (0, 0)),
            pl.BlockSpec((1, W1.shape[1]), lambda i: (0, 0)),
            pl.BlockSpec((W1.shape[1], M), lambda i: (0, 0)),
            pl.BlockSpec((1, M), lambda i: (0, 0)),
        ],
        out_specs=pl.BlockSpec((block, M), lambda i: (i, 0)),
        out_shape=jax.ShapeDtypeStruct((N, M), jnp.bfloat16),
    )(t, W1, b1, W2, b2)


def _final_mm(u32, W, b, block):
    N = u32.shape[0]
    M = W.shape[1]

    def body(ur, wer, wor, br, outr):
        a = ur[...]
        lo = jax.lax.bitcast_convert_type(a << 16, jnp.float32)
        hi = jax.lax.bitcast_convert_type(a & jnp.int32(-65536), jnp.float32)
        d = (jnp.dot(jnp.maximum(lo, 0.0), wer[...],
                     preferred_element_type=jnp.float32)
             + jnp.dot(jnp.maximum(hi, 0.0), wor[...],
                       preferred_element_type=jnp.float32))
        outr[...] = EPS * (d + br[...])

    K = u32.shape[1]
    return pl.pallas_call(
        body,
        grid=(N // block,),
        in_specs=[
            pl.BlockSpec((block, K), lambda i: (i, 0)),
            pl.BlockSpec((K, M), lambda i: (0, 0)),
            pl.BlockSpec((K, M), lambda i: (0, 0)),
            pl.BlockSpec((1, M), lambda i: (0, 0)),
        ],
        out_specs=pl.BlockSpec((block, M), lambda i: (i, 0)),
        out_shape=jax.ShapeDtypeStruct((N, M), jnp.float32),
    )(u32, W[0::2], W[1::2], b)


def _gather_add(table, src, dst, n_edges, chunk, n_main, tail_chunk, n_tail):
    dt = table.dtype
    width = table.shape[1]
    per_w = n_main * chunk
    tail_base = NW * per_w
    mesh = plsc.VectorSubcoreMesh(core_axis_name="c", subcore_axis_name="s")

    @functools.partial(
        pl.kernel,
        out_type=jax.ShapeDtypeStruct((n_edges, width), dt),
        mesh=mesh,
        scratch_types=[
            pltpu.VMEM((per_w,), jnp.int32),
            pltpu.VMEM((per_w,), jnp.int32),
            pltpu.VMEM((tail_chunk,), jnp.int32),
            pltpu.VMEM((tail_chunk,), jnp.int32),
            pltpu.VMEM((chunk, width), dt),
            pltpu.VMEM((chunk, width), dt),
            pltpu.VMEM((chunk, width), dt),
            pltpu.VMEM((chunk, width), dt),
            pltpu.SemaphoreType.DMA,
            pltpu.SemaphoreType.DMA,
            pltpu.SemaphoreType.DMA,
            pltpu.SemaphoreType.DMA,
            pltpu.SemaphoreType.DMA,
            pltpu.SemaphoreType.DMA,
        ],
    )
    def k(table_hbm, src_hbm, dst_hbm, out_hbm, sidx, didx, tsidx, tdidx,
          rowsa0, rowsa1, rowsb0, rowsb1, ga0, ga1, gb0, gb1, ws0, ws1):
        wid = lax.axis_index("s") * NC + lax.axis_index("c")
        base = wid * per_w
        rowsa = (rowsa0, rowsa1)
        rowsb = (rowsb0, rowsb1)
        ga = (ga0, ga1)
        gb = (gb0, gb1)
        ws = (ws0, ws1)

        def add_rows(ba, bb, n_rows):
            def add_row(r, carry):
                for j in range(width // 16):
                    sl = pl.ds(j * 16, 16)
                    a = ba[r, sl]
                    b_ = bb[r, sl]
                    if dt == jnp.int32:
                        bc = jax.lax.bitcast_convert_type
                        alo = bc(a << 16, jnp.float32)
                        ahi = bc(a & MASK_HI, jnp.float32)
                        blo = bc(b_ << 16, jnp.float32)
                        bhi = bc(b_ & MASK_HI, jnp.float32)
                        ilo = bc(alo + blo, jnp.int32)
                        ihi = bc(ahi + bhi, jnp.int32)
                        rlo = ilo + 0x7FFF + ((ilo >> 16) & 1)
                        rhi = ihi + 0x7FFF + ((ihi >> 16) & 1)
                        ba[r, sl] = ((rhi & MASK_HI)
                                     | ((rlo >> 16) & MASK_LO))
                    else:
                        ba[r, sl] = a + b_
                return carry
            lax.fori_loop(0, n_rows, add_row, 0)

        pltpu.sync_copy(src_hbm.at[pl.ds(base, per_w)], sidx)
        pltpu.sync_copy(dst_hbm.at[pl.ds(base, per_w)], didx)

        @pl.when(wid < n_tail)
        def _():
            toff = tail_base + wid * tail_chunk
            pltpu.sync_copy(src_hbm.at[pl.ds(toff, tail_chunk)], tsidx)
            pltpu.sync_copy(dst_hbm.at[pl.ds(toff, tail_chunk)], tdidx)
            tra = rowsa0.at[pl.ds(0, tail_chunk)]
            trb = rowsb0.at[pl.ds(0, tail_chunk)]
            pltpu.async_copy(table_hbm.at[tsidx], tra, ga0)
            pltpu.async_copy(table_hbm.at[tdidx], trb, gb0)
            pltpu.make_async_copy(
                table_hbm.at[pl.ds(0, tail_chunk)], tra, ga0).wait()
            pltpu.make_async_copy(
                table_hbm.at[pl.ds(0, tail_chunk)], trb, gb0).wait()
            add_rows(rowsa0, rowsb0, tail_chunk)
            pltpu.sync_copy(tra, out_hbm.at[pl.ds(toff, tail_chunk)])

        pltpu.async_copy(table_hbm.at[sidx.at[pl.ds(0, chunk)]], rowsa0, ga0)
        pltpu.async_copy(table_hbm.at[didx.at[pl.ds(0, chunk)]], rowsb0, gb0)

        def step(c, b, nb):
            pltpu.make_async_copy(
                table_hbm.at[pl.ds(0, chunk)], rowsa[b], ga[b]).wait()
            pltpu.make_async_copy(
                table_hbm.at[pl.ds(0, chunk)], rowsb[b], gb[b]).wait()
            @pl.when(c + 1 < n_main)
            def _():
                @pl.when(c >= 1)
                def _():
                    pltpu.make_async_copy(
                        rowsa[nb], out_hbm.at[pl.ds(0, chunk)], ws[nb]).wait()
                pltpu.async_copy(
                    table_hbm.at[sidx.at[pl.ds((c + 1) * chunk, chunk)]],
                    rowsa[nb], ga[nb])
                pltpu.async_copy(
                    table_hbm.at[didx.at[pl.ds((c + 1) * chunk, chunk)]],
                    rowsb[nb], gb[nb])
            add_rows(rowsa[b], rowsb[b], chunk)
            pltpu.async_copy(
                rowsa[b], out_hbm.at[pl.ds(base + c * chunk, chunk)], ws[b])

        def pair(c2, carry):
            c = c2 * 2
            step(c, 0, 1)
            @pl.when(c + 1 < n_main)
            def _():
                step(c + 1, 1, 0)
            return carry

        lax.fori_loop(0, (n_main + 1) // 2, pair, 0)

        pltpu.make_async_copy(rowsa0, out_hbm.at[pl.ds(0, chunk)], ws0).wait()
        pltpu.make_async_copy(rowsa1, out_hbm.at[pl.ds(0, chunk)], ws1).wait()

    return k(table, src, dst)


def kernel(x, edge_index, W1a, b1a, W1b, b1b, W2a, b2a, W2b, b2b):
    n_edges = edge_index.shape[1]
    src = edge_index[0].astype(jnp.int32)
    dst = edge_index[1].astype(jnp.int32)

    q = _mm_bias(x, W1a, b1a[None, :], block=1000)
    t = _gather_add(q, src, dst, N_NODES,
                    chunk=104, n_main=3, tail_chunk=16, n_tail=1)
    p = _mid_chain(t, W1b, b1b[None, :],
                   W2a, b2a[None, :], block=1000)
    p32 = jax.lax.bitcast_convert_type(
        p.reshape(N_NODES, D // 2, 2), jnp.int32)
    u32 = _gather_add(p32, src, dst, n_edges,
                      chunk=104, n_main=48, tail_chunk=64, n_tail=4)
    return _final_mm(u32, W2b, b2b[None, :], block=8000)


# speedup vs baseline: 1.3020x; 1.3020x over previous
import functools

import jax
import jax.numpy as jnp
from jax import lax
from jax.experimental import pallas as pl
from jax.experimental.pallas import tpu as pltpu
from jax.experimental.pallas import tpu_sc as plsc

N_NODES = 10000
D = 256
EPS = 0.5
NC = 2
NS = 16
NW = NC * NS
MASK_HI = -65536
MASK_LO = 0xFFFF


def _mm_bias(x, W, b, block):
    N, K = x.shape
    M = W.shape[1]
    return pl.pallas_call(
        lambda xr, wr, br, outr: outr.__setitem__(
            ..., jnp.dot(xr[...], wr[...], preferred_element_type=jnp.float32)
            + 0.5 * br[...]),
        grid=(N // block,),
        in_specs=[
            pl.BlockSpec((block, K), lambda i: (i, 0)),
            pl.BlockSpec((K, M), lambda i: (0, 0)),
            pl.BlockSpec((1, M), lambda i: (0, 0)),
        ],
        out_specs=pl.BlockSpec((block, M), lambda i: (i, 0)),
        out_shape=jax.ShapeDtypeStruct((N, M), jnp.float32),
    )(x, W, b)


def _mid_chain(t, W1, b1, We, Wo, be, bo, block):
    N, K = t.shape
    M = We.shape[1]

    def rne16(x):
        i = jax.lax.bitcast_convert_type(x, jnp.int32)
        return i + 0x7FFF + ((i >> 16) & 1)

    def body(tr, w1r, b1r, wer, wor, ber, bor, outr):
        h = jnp.maximum(tr[...], 0.0)
        h = jnp.dot(h, w1r[...], preferred_element_type=jnp.float32) + b1r[...]
        h = jnp.maximum(EPS * h, 0.0)
        pe = (jnp.dot(h, wer[...], preferred_element_type=jnp.float32)
              + 0.5 * ber[...])
        po = (jnp.dot(h, wor[...], preferred_element_type=jnp.float32)
              + 0.5 * bor[...])
        outr[...] = ((rne16(po) & MASK_HI)
                     | ((rne16(pe) >> 16) & MASK_LO))

    return pl.pallas_call(
        body,
        grid=(N // block,),
        in_specs=[
            pl.BlockSpec((block, K), lambda i: (i, 0)),
            pl.BlockSpec((K, W1.shape[1]), lambda i: (0, 0)),
            pl.BlockSpec((1, W1.shape[1]), lambda i: (0, 0)),
            pl.BlockSpec((W1.shape[1], M), lambda i: (0, 0)),
            pl.BlockSpec((W1.shape[1], M), lambda i: (0, 0)),
            pl.BlockSpec((1, M), lambda i: (0, 0)),
            pl.BlockSpec((1, M), lambda i: (0, 0)),
        ],
        out_specs=pl.BlockSpec((block, M), lambda i: (i, 0)),
        out_shape=jax.ShapeDtypeStruct((N, M), jnp.int32),
    )(t, W1, b1, We, Wo, be, bo)


def _final_mm(u32, W, b, block):
    N = u32.shape[0]
    M = W.shape[1]

    def body(ur, wer, wor, br, outr):
        a = ur[...]
        lo = jax.lax.bitcast_convert_type(a << 16, jnp.float32)
        hi = jax.lax.bitcast_convert_type(a & jnp.int32(-65536), jnp.float32)
        d = (jnp.dot(jnp.maximum(lo, 0.0), wer[...],
                     preferred_element_type=jnp.float32)
             + jnp.dot(jnp.maximum(hi, 0.0), wor[...],
                       preferred_element_type=jnp.float32))
        outr[...] = EPS * (d + br[...])

    K = u32.shape[1]
    return pl.pallas_call(
        body,
        grid=(N // block,),
        in_specs=[
            pl.BlockSpec((block, K), lambda i: (i, 0)),
            pl.BlockSpec((K, M), lambda i: (0, 0)),
            pl.BlockSpec((K, M), lambda i: (0, 0)),
            pl.BlockSpec((1, M), lambda i: (0, 0)),
        ],
        out_specs=pl.BlockSpec((block, M), lambda i: (i, 0)),
        out_shape=jax.ShapeDtypeStruct((N, M), jnp.float32),
    )(u32, W[0::2], W[1::2], b)


def _gather_add(table, src, dst, n_edges, chunk, n_main, tail_chunk, n_tail):
    dt = table.dtype
    width = table.shape[1]
    per_w = n_main * chunk
    tail_base = NW * per_w
    mesh = plsc.VectorSubcoreMesh(core_axis_name="c", subcore_axis_name="s")

    @functools.partial(
        pl.kernel,
        out_type=jax.ShapeDtypeStruct((n_edges, width), dt),
        mesh=mesh,
        scratch_types=[
            pltpu.VMEM((per_w,), jnp.int32),
            pltpu.VMEM((per_w,), jnp.int32),
            pltpu.VMEM((tail_chunk,), jnp.int32),
            pltpu.VMEM((tail_chunk,), jnp.int32),
            pltpu.VMEM((chunk, width), dt),
            pltpu.VMEM((chunk, width), dt),
            pltpu.VMEM((chunk, width), dt),
            pltpu.VMEM((chunk, width), dt),
            pltpu.SemaphoreType.DMA,
            pltpu.SemaphoreType.DMA,
            pltpu.SemaphoreType.DMA,
            pltpu.SemaphoreType.DMA,
            pltpu.SemaphoreType.DMA,
            pltpu.SemaphoreType.DMA,
        ],
    )
    def k(table_hbm, src_hbm, dst_hbm, out_hbm, sidx, didx, tsidx, tdidx,
          rowsa0, rowsa1, rowsb0, rowsb1, ga0, ga1, gb0, gb1, ws0, ws1):
        wid = lax.axis_index("s") * NC + lax.axis_index("c")
        base = wid * per_w
        rowsa = (rowsa0, rowsa1)
        rowsb = (rowsb0, rowsb1)
        ga = (ga0, ga1)
        gb = (gb0, gb1)
        ws = (ws0, ws1)

        def add_rows(ba, bb, n_rows):
            def add_row(r, carry):
                for j in range(width // 16):
                    sl = pl.ds(j * 16, 16)
                    a = ba[r, sl]
                    b_ = bb[r, sl]
                    if dt == jnp.int32:
                        bc = jax.lax.bitcast_convert_type
                        alo = bc(a << 16, jnp.float32)
                        ahi = bc(a & MASK_HI, jnp.float32)
                        blo = bc(b_ << 16, jnp.float32)
                        bhi = bc(b_ & MASK_HI, jnp.float32)
                        ilo = bc(alo + blo, jnp.int32)
                        ihi = bc(ahi + bhi, jnp.int32)
                        rlo = ilo + 0x7FFF + ((ilo >> 16) & 1)
                        rhi = ihi + 0x7FFF + ((ihi >> 16) & 1)
                        ba[r, sl] = ((rhi & MASK_HI)
                                     | ((rlo >> 16) & MASK_LO))
                    else:
                        ba[r, sl] = a + b_
                return carry
            lax.fori_loop(0, n_rows, add_row, 0)

        pltpu.sync_copy(src_hbm.at[pl.ds(base, per_w)], sidx)
        pltpu.sync_copy(dst_hbm.at[pl.ds(base, per_w)], didx)

        @pl.when(wid < n_tail)
        def _():
            toff = tail_base + wid * tail_chunk
            pltpu.sync_copy(src_hbm.at[pl.ds(toff, tail_chunk)], tsidx)
            pltpu.sync_copy(dst_hbm.at[pl.ds(toff, tail_chunk)], tdidx)
            tra = rowsa0.at[pl.ds(0, tail_chunk)]
            trb = rowsb0.at[pl.ds(0, tail_chunk)]
            pltpu.async_copy(table_hbm.at[tsidx], tra, ga0)
            pltpu.async_copy(table_hbm.at[tdidx], trb, gb0)
            pltpu.make_async_copy(
                table_hbm.at[pl.ds(0, tail_chunk)], tra, ga0).wait()
            pltpu.make_async_copy(
                table_hbm.at[pl.ds(0, tail_chunk)], trb, gb0).wait()
            add_rows(rowsa0, rowsb0, tail_chunk)
            pltpu.sync_copy(tra, out_hbm.at[pl.ds(toff, tail_chunk)])

        pltpu.async_copy(table_hbm.at[sidx.at[pl.ds(0, chunk)]], rowsa0, ga0)
        pltpu.async_copy(table_hbm.at[didx.at[pl.ds(0, chunk)]], rowsb0, gb0)

        def step(c, b, nb):
            pltpu.make_async_copy(
                table_hbm.at[pl.ds(0, chunk)], rowsa[b], ga[b]).wait()
            pltpu.make_async_copy(
                table_hbm.at[pl.ds(0, chunk)], rowsb[b], gb[b]).wait()
            @pl.when(c + 1 < n_main)
            def _():
                @pl.when(c >= 1)
                def _():
                    pltpu.make_async_copy(
                        rowsa[nb], out_hbm.at[pl.ds(0, chunk)], ws[nb]).wait()
                pltpu.async_copy(
                    table_hbm.at[sidx.at[pl.ds((c + 1) * chunk, chunk)]],
                    rowsa[nb], ga[nb])
                pltpu.async_copy(
                    table_hbm.at[didx.at[pl.ds((c + 1) * chunk, chunk)]],
                    rowsb[nb], gb[nb])
            add_rows(rowsa[b], rowsb[b], chunk)
            pltpu.async_copy(
                rowsa[b], out_hbm.at[pl.ds(base + c * chunk, chunk)], ws[b])

        def pair(c2, carry):
            c = c2 * 2
            step(c, 0, 1)
            @pl.when(c + 1 < n_main)
            def _():
                step(c + 1, 1, 0)
            return carry

        lax.fori_loop(0, (n_main + 1) // 2, pair, 0)

        pltpu.make_async_copy(rowsa0, out_hbm.at[pl.ds(0, chunk)], ws0).wait()
        pltpu.make_async_copy(rowsa1, out_hbm.at[pl.ds(0, chunk)], ws1).wait()

    return k(table, src, dst)


def kernel(x, edge_index, W1a, b1a, W1b, b1b, W2a, b2a, W2b, b2b):
    n_edges = edge_index.shape[1]
    src = edge_index[0].astype(jnp.int32)
    dst = edge_index[1].astype(jnp.int32)

    q = _mm_bias(x, W1a, b1a[None, :], block=1000)
    t = _gather_add(q, src, dst, N_NODES,
                    chunk=104, n_main=3, tail_chunk=16, n_tail=1)
    p32 = _mid_chain(t, W1b, b1b[None, :],
                     W2a[:, 0::2], W2a[:, 1::2],
                     b2a[None, 0::2], b2a[None, 1::2],
                     block=1000)
    u32 = _gather_add(p32, src, dst, n_edges,
                      chunk=104, n_main=48, tail_chunk=64, n_tail=4)
    return _final_mm(u32, W2b, b2b[None, :], block=8000)


# speedup vs baseline: 1.5073x; 1.1576x over previous
import functools

import jax
import jax.numpy as jnp
from jax import lax
from jax.experimental import pallas as pl
from jax.experimental.pallas import tpu as pltpu
from jax.experimental.pallas import tpu_sc as plsc

N_NODES = 10000
D = 256
EPS = 0.5
NC = 2
NS = 16
NW = NC * NS
MASK_HI = -65536
MASK_LO = 0xFFFF


def _mm_bias(x, W, b, block):
    N, K = x.shape
    M = W.shape[1]
    return pl.pallas_call(
        lambda xr, wr, br, outr: outr.__setitem__(
            ..., jnp.dot(xr[...], wr[...], preferred_element_type=jnp.float32)
            + 0.5 * br[...]),
        grid=(N // block,),
        in_specs=[
            pl.BlockSpec((block, K), lambda i: (i, 0)),
            pl.BlockSpec((K, M), lambda i: (0, 0)),
            pl.BlockSpec((1, M), lambda i: (0, 0)),
        ],
        out_specs=pl.BlockSpec((block, M), lambda i: (i, 0)),
        out_shape=jax.ShapeDtypeStruct((N, M), jnp.float32),
    )(x, W, b)


def _mid_chain(t, W1, b1, We, Wo, be, bo, block):
    N, K = t.shape
    M = We.shape[1]

    def rne16(x):
        i = jax.lax.bitcast_convert_type(x, jnp.int32)
        return i + 0x7FFF + ((i >> 16) & 1)

    def body(tr, w1r, b1r, wer, wor, ber, bor, outr):
        h = jnp.maximum(tr[...], 0.0)
        h = jnp.dot(h, w1r[...], preferred_element_type=jnp.float32) + b1r[...]
        h = jnp.maximum(EPS * h, 0.0)
        pe = (jnp.dot(h, wer[...], preferred_element_type=jnp.float32)
              + 0.5 * ber[...])
        po = (jnp.dot(h, wor[...], preferred_element_type=jnp.float32)
              + 0.5 * bor[...])
        outr[...] = ((rne16(po) & MASK_HI)
                     | ((rne16(pe) >> 16) & MASK_LO))

    return pl.pallas_call(
        body,
        grid=(N // block,),
        in_specs=[
            pl.BlockSpec((block, K), lambda i: (i, 0)),
            pl.BlockSpec((K, W1.shape[1]), lambda i: (0, 0)),
            pl.BlockSpec((1, W1.shape[1]), lambda i: (0, 0)),
            pl.BlockSpec((W1.shape[1], M), lambda i: (0, 0)),
            pl.BlockSpec((W1.shape[1], M), lambda i: (0, 0)),
            pl.BlockSpec((1, M), lambda i: (0, 0)),
            pl.BlockSpec((1, M), lambda i: (0, 0)),
        ],
        out_specs=pl.BlockSpec((block, M), lambda i: (i, 0)),
        out_shape=jax.ShapeDtypeStruct((N, M), jnp.int32),
    )(t, W1, b1, We, Wo, be, bo)


def _final_mm(u32, W, b, block):
    N = u32.shape[0]
    M = W.shape[1]

    def body(ur, wer, wor, br, outr):
        a = ur[...]
        lo = jax.lax.bitcast_convert_type(a << 16, jnp.float32)
        hi = jax.lax.bitcast_convert_type(a & jnp.int32(-65536), jnp.float32)
        d = (jnp.dot(jnp.maximum(lo, 0.0), wer[...],
                     preferred_element_type=jnp.float32)
             + jnp.dot(jnp.maximum(hi, 0.0), wor[...],
                       preferred_element_type=jnp.float32))
        outr[...] = EPS * (d + br[...])

    K = u32.shape[1]
    return pl.pallas_call(
        body,
        grid=(N // block,),
        in_specs=[
            pl.BlockSpec((block, K), lambda i: (i, 0)),
            pl.BlockSpec((K, M), lambda i: (0, 0)),
            pl.BlockSpec((K, M), lambda i: (0, 0)),
            pl.BlockSpec((1, M), lambda i: (0, 0)),
        ],
        out_specs=pl.BlockSpec((block, M), lambda i: (i, 0)),
        out_shape=jax.ShapeDtypeStruct((N, M), jnp.float32),
    )(u32, W[0::2], W[1::2], b)


def _gather_add(table, src, dst, n_edges, chunk, n_main, tail_chunk, n_tail):
    dt = table.dtype
    width = table.shape[1]
    per_w = n_main * chunk
    tail_base = NW * per_w
    mesh = plsc.VectorSubcoreMesh(core_axis_name="c", subcore_axis_name="s")

    @functools.partial(
        pl.kernel,
        out_type=jax.ShapeDtypeStruct((n_edges, width), dt),
        mesh=mesh,
        scratch_types=[
            pltpu.VMEM((per_w,), jnp.int32),
            pltpu.VMEM((per_w,), jnp.int32),
            pltpu.VMEM((tail_chunk,), jnp.int32),
            pltpu.VMEM((tail_chunk,), jnp.int32),
            pltpu.VMEM((chunk, width), dt),
            pltpu.VMEM((chunk, width), dt),
            pltpu.VMEM((chunk, width), dt),
            pltpu.VMEM((chunk, width), dt),
            pltpu.SemaphoreType.DMA,
            pltpu.SemaphoreType.DMA,
            pltpu.SemaphoreType.DMA,
            pltpu.SemaphoreType.DMA,
            pltpu.SemaphoreType.DMA,
            pltpu.SemaphoreType.DMA,
        ],
    )
    def k(table_hbm, src_hbm, dst_hbm, out_hbm, sidx, didx, tsidx, tdidx,
          rowsa0, rowsa1, rowsb0, rowsb1, ga0, ga1, gb0, gb1, ws0, ws1):
        wid = lax.axis_index("s") * NC + lax.axis_index("c")
        base = wid * per_w
        rowsa = (rowsa0, rowsa1)
        rowsb = (rowsb0, rowsb1)
        ga = (ga0, ga1)
        gb = (gb0, gb1)
        ws = (ws0, ws1)

        def add_rows(ba, bb, n_rows):
            def add_row(r, carry):
                for j in range(width // 16):
                    sl = pl.ds(j * 16, 16)
                    a = ba[r, sl]
                    b_ = bb[r, sl]
                    if dt == jnp.int32:
                        bc = jax.lax.bitcast_convert_type
                        alo = bc(a << 16, jnp.float32)
                        ahi = bc(a & MASK_HI, jnp.float32)
                        blo = bc(b_ << 16, jnp.float32)
                        bhi = bc(b_ & MASK_HI, jnp.float32)
                        rlo = bc(alo + blo, jnp.int32) + 0x8000
                        rhi = bc(ahi + bhi, jnp.int32) + 0x8000
                        ba[r, sl] = ((rhi & MASK_HI)
                                     | ((rlo >> 16) & MASK_LO))
                    else:
                        ba[r, sl] = a + b_
                return carry
            lax.fori_loop(0, n_rows, add_row, 0)

        pltpu.sync_copy(src_hbm.at[pl.ds(base, per_w)], sidx)
        pltpu.sync_copy(dst_hbm.at[pl.ds(base, per_w)], didx)

        @pl.when(wid < n_tail)
        def _():
            toff = tail_base + wid * tail_chunk
            pltpu.sync_copy(src_hbm.at[pl.ds(toff, tail_chunk)], tsidx)
            pltpu.sync_copy(dst_hbm.at[pl.ds(toff, tail_chunk)], tdidx)
            tra = rowsa0.at[pl.ds(0, tail_chunk)]
            trb = rowsb0.at[pl.ds(0, tail_chunk)]
            pltpu.async_copy(table_hbm.at[tsidx], tra, ga0)
            pltpu.async_copy(table_hbm.at[tdidx], trb, gb0)
            pltpu.make_async_copy(
                table_hbm.at[pl.ds(0, tail_chunk)], tra, ga0).wait()
            pltpu.make_async_copy(
                table_hbm.at[pl.ds(0, tail_chunk)], trb, gb0).wait()
            add_rows(rowsa0, rowsb0, tail_chunk)
            pltpu.sync_copy(tra, out_hbm.at[pl.ds(toff, tail_chunk)])

        pltpu.async_copy(table_hbm.at[sidx.at[pl.ds(0, chunk)]], rowsa0, ga0)
        pltpu.async_copy(table_hbm.at[didx.at[pl.ds(0, chunk)]], rowsb0, gb0)

        def step(c, b, nb):
            pltpu.make_async_copy(
                table_hbm.at[pl.ds(0, chunk)], rowsa[b], ga[b]).wait()
            pltpu.make_async_copy(
                table_hbm.at[pl.ds(0, chunk)], rowsb[b], gb[b]).wait()
            @pl.when(c + 1 < n_main)
            def _():
                @pl.when(c >= 1)
                def _():
                    pltpu.make_async_copy(
                        rowsa[nb], out_hbm.at[pl.ds(0, chunk)], ws[nb]).wait()
                pltpu.async_copy(
                    table_hbm.at[sidx.at[pl.ds((c + 1) * chunk, chunk)]],
                    rowsa[nb], ga[nb])
                pltpu.async_copy(
                    table_hbm.at[didx.at[pl.ds((c + 1) * chunk, chunk)]],
                    rowsb[nb], gb[nb])
            add_rows(rowsa[b], rowsb[b], chunk)
            pltpu.async_copy(
                rowsa[b], out_hbm.at[pl.ds(base + c * chunk, chunk)], ws[b])

        def pair(c2, carry):
            c = c2 * 2
            step(c, 0, 1)
            @pl.when(c + 1 < n_main)
            def _():
                step(c + 1, 1, 0)
            return carry

        lax.fori_loop(0, (n_main + 1) // 2, pair, 0)

        pltpu.make_async_copy(rowsa0, out_hbm.at[pl.ds(0, chunk)], ws0).wait()
        pltpu.make_async_copy(rowsa1, out_hbm.at[pl.ds(0, chunk)], ws1).wait()

    return k(table, src, dst)


def kernel(x, edge_index, W1a, b1a, W1b, b1b, W2a, b2a, W2b, b2b):
    n_edges = edge_index.shape[1]
    src = edge_index[0].astype(jnp.int32)
    dst = edge_index[1].astype(jnp.int32)

    q = _mm_bias(x, W1a, b1a[None, :], block=1000)
    t = _gather_add(q, src, dst, N_NODES,
                    chunk=104, n_main=3, tail_chunk=16, n_tail=1)
    p32 = _mid_chain(t, W1b, b1b[None, :],
                     W2a[:, 0::2], W2a[:, 1::2],
                     b2a[None, 0::2], b2a[None, 1::2],
                     block=1000)
    u32 = _gather_add(p32, src, dst, n_edges,
                      chunk=104, n_main=48, tail_chunk=64, n_tail=4)
    return _final_mm(u32, W2b, b2b[None, :], block=8000)


# speedup vs baseline: 1.8615x; 1.2350x over previous
import functools

import jax
import jax.numpy as jnp
from jax import lax
from jax.experimental import pallas as pl
from jax.experimental.pallas import tpu as pltpu
from jax.experimental.pallas import tpu_sc as plsc

N_NODES = 10000
D = 256
EPS = 0.5
NC = 2
NS = 16
NW = NC * NS
MASK_HI = -65536
MASK_LO = 0xFFFF


def _mm_bias(x, W, b, block):
    N, K = x.shape
    M = W.shape[1]
    return pl.pallas_call(
        lambda xr, wr, br, outr: outr.__setitem__(
            ..., jnp.dot(xr[...], wr[...], preferred_element_type=jnp.float32)
            + 0.5 * br[...]),
        grid=(N // block,),
        in_specs=[
            pl.BlockSpec((block, K), lambda i: (i, 0)),
            pl.BlockSpec((K, M), lambda i: (0, 0)),
            pl.BlockSpec((1, M), lambda i: (0, 0)),
        ],
        out_specs=pl.BlockSpec((block, M), lambda i: (i, 0)),
        out_shape=jax.ShapeDtypeStruct((N, M), jnp.float32),
    )(x, W, b)


def _mid_chain(t, W1, b1, W2, b2, block):
    N, K = t.shape
    M = W2.shape[1] // 2

    def rhu16(x):
        return jax.lax.bitcast_convert_type(x, jnp.int32) + 0x8000

    def body(tr, w1r, b1r, w2r, b2r, outr):
        h = jnp.maximum(tr[...], 0.0)
        h = jnp.dot(h, w1r[...], preferred_element_type=jnp.float32) + b1r[...]
        h = jnp.maximum(EPS * h, 0.0)
        w2 = w2r[...]
        b2 = b2r[...]
        pe = (jnp.dot(h, w2[:, :M], preferred_element_type=jnp.float32)
              + 0.5 * b2[:, :M])
        po = (jnp.dot(h, w2[:, M:], preferred_element_type=jnp.float32)
              + 0.5 * b2[:, M:])
        outr[...] = ((rhu16(po) & MASK_HI)
                     | ((rhu16(pe) >> 16) & MASK_LO))

    return pl.pallas_call(
        body,
        grid=(N // block,),
        in_specs=[
            pl.BlockSpec((block, K), lambda i: (i, 0)),
            pl.BlockSpec((K, W1.shape[1]), lambda i: (0, 0)),
            pl.BlockSpec((1, W1.shape[1]), lambda i: (0, 0)),
            pl.BlockSpec((W2.shape[0], 2 * M), lambda i: (0, 0)),
            pl.BlockSpec((1, 2 * M), lambda i: (0, 0)),
        ],
        out_specs=pl.BlockSpec((block, M), lambda i: (i, 0)),
        out_shape=jax.ShapeDtypeStruct((N, M), jnp.int32),
    )(t, W1, b1, W2, b2)


def _final_mm(u32, W, b, block):
    N = u32.shape[0]
    K2, M = W.shape
    K = K2 // 2

    def body(ur, wr, br, outr):
        a = ur[...]
        lo = jax.lax.bitcast_convert_type(a << 16, jnp.float32)
        hi = jax.lax.bitcast_convert_type(a & MASK_HI, jnp.float32)
        w = wr[...]
        dn = (((0,), (1,)), ((), ()))
        d = (jax.lax.dot_general(w[:K], jnp.maximum(lo, 0.0), dn,
                                 preferred_element_type=jnp.float32)
             + jax.lax.dot_general(w[K:], jnp.maximum(hi, 0.0), dn,
                                   preferred_element_type=jnp.float32))
        outr[...] = EPS * (d + br[...])

    out_t = pl.pallas_call(
        body,
        grid=(N // block,),
        in_specs=[
            pl.BlockSpec((block, K), lambda i: (i, 0)),
            pl.BlockSpec((K2, M), lambda i: (0, 0)),
            pl.BlockSpec((M, 1), lambda i: (0, 0)),
        ],
        out_specs=pl.BlockSpec((M, block), lambda i: (0, i)),
        out_shape=jax.ShapeDtypeStruct((M, N), jnp.float32),
    )(u32, W, b)
    return out_t.T


def _gather_add(table, src, dst, n_edges, chunk, n_main, tail_chunk, n_tail):
    dt = table.dtype
    width = table.shape[1]
    per_w = n_main * chunk
    tail_base = NW * per_w
    mesh = plsc.VectorSubcoreMesh(core_axis_name="c", subcore_axis_name="s")

    @functools.partial(
        pl.kernel,
        out_type=jax.ShapeDtypeStruct((n_edges, width), dt),
        mesh=mesh,
        scratch_types=[
            pltpu.VMEM((per_w,), jnp.int32),
            pltpu.VMEM((per_w,), jnp.int32),
            pltpu.VMEM((tail_chunk,), jnp.int32),
            pltpu.VMEM((tail_chunk,), jnp.int32),
            pltpu.VMEM((chunk, width), dt),
            pltpu.VMEM((chunk, width), dt),
            pltpu.VMEM((chunk, width), dt),
            pltpu.VMEM((chunk, width), dt),
            pltpu.SemaphoreType.DMA,
            pltpu.SemaphoreType.DMA,
            pltpu.SemaphoreType.DMA,
            pltpu.SemaphoreType.DMA,
            pltpu.SemaphoreType.DMA,
            pltpu.SemaphoreType.DMA,
        ],
    )
    def k(table_hbm, src_hbm, dst_hbm, out_hbm, sidx, didx, tsidx, tdidx,
          rowsa0, rowsa1, rowsb0, rowsb1, ga0, ga1, gb0, gb1, ws0, ws1):
        wid = lax.axis_index("s") * NC + lax.axis_index("c")
        base = wid * per_w
        rowsa = (rowsa0, rowsa1)
        rowsb = (rowsb0, rowsb1)
        ga = (ga0, ga1)
        gb = (gb0, gb1)
        ws = (ws0, ws1)

        def add_rows(ba, bb, n_rows):
            def add_row(r, carry):
                for j in range(width // 16):
                    sl = pl.ds(j * 16, 16)
                    a = ba[r, sl]
                    b_ = bb[r, sl]
                    if dt == jnp.int32:
                        bc = jax.lax.bitcast_convert_type
                        alo = bc(a << 16, jnp.float32)
                        ahi = bc(a & MASK_HI, jnp.float32)
                        blo = bc(b_ << 16, jnp.float32)
                        bhi = bc(b_ & MASK_HI, jnp.float32)
                        rlo = bc(alo + blo, jnp.int32) + 0x8000
                        rhi = bc(ahi + bhi, jnp.int32) + 0x8000
                        ba[r, sl] = ((rhi & MASK_HI)
                                     | ((rlo >> 16) & MASK_LO))
                    else:
                        ba[r, sl] = a + b_
                return carry
            lax.fori_loop(0, n_rows, add_row, 0)

        pltpu.sync_copy(src_hbm.at[pl.ds(base, per_w)], sidx)
        pltpu.sync_copy(dst_hbm.at[pl.ds(base, per_w)], didx)

        @pl.when(wid < n_tail)
        def _():
            toff = tail_base + wid * tail_chunk
            pltpu.sync_copy(src_hbm.at[pl.ds(toff, tail_chunk)], tsidx)
            pltpu.sync_copy(dst_hbm.at[pl.ds(toff, tail_chunk)], tdidx)
            tra = rowsa0.at[pl.ds(0, tail_chunk)]
            trb = rowsb0.at[pl.ds(0, tail_chunk)]
            pltpu.async_copy(table_hbm.at[tsidx], tra, ga0)
            pltpu.async_copy(table_hbm.at[tdidx], trb, gb0)
            pltpu.make_async_copy(
                table_hbm.at[pl.ds(0, tail_chunk)], tra, ga0).wait()
            pltpu.make_async_copy(
                table_hbm.at[pl.ds(0, tail_chunk)], trb, gb0).wait()
            add_rows(rowsa0, rowsb0, tail_chunk)
            pltpu.sync_copy(tra, out_hbm.at[pl.ds(toff, tail_chunk)])

        pltpu.async_copy(table_hbm.at[sidx.at[pl.ds(0, chunk)]], rowsa0, ga0)
        pltpu.async_copy(table_hbm.at[didx.at[pl.ds(0, chunk)]], rowsb0, gb0)

        def step(c, b, nb):
            pltpu.make_async_copy(
                table_hbm.at[pl.ds(0, chunk)], rowsa[b], ga[b]).wait()
            pltpu.make_async_copy(
                table_hbm.at[pl.ds(0, chunk)], rowsb[b], gb[b]).wait()
            @pl.when(c + 1 < n_main)
            def _():
                @pl.when(c >= 1)
                def _():
                    pltpu.make_async_copy(
                        rowsa[nb], out_hbm.at[pl.ds(0, chunk)], ws[nb]).wait()
                pltpu.async_copy(
                    table_hbm.at[sidx.at[pl.ds((c + 1) * chunk, chunk)]],
                    rowsa[nb], ga[nb])
                pltpu.async_copy(
                    table_hbm.at[didx.at[pl.ds((c + 1) * chunk, chunk)]],
                    rowsb[nb], gb[nb])
            add_rows(rowsa[b], rowsb[b], chunk)
            pltpu.async_copy(
                rowsa[b], out_hbm.at[pl.ds(base + c * chunk, chunk)], ws[b])

        def pair(c2, carry):
            c = c2 * 2
            step(c, 0, 1)
            @pl.when(c + 1 < n_main)
            def _():
                step(c + 1, 1, 0)
            return carry

        lax.fori_loop(0, (n_main + 1) // 2, pair, 0)

        pltpu.make_async_copy(rowsa0, out_hbm.at[pl.ds(0, chunk)], ws0).wait()
        pltpu.make_async_copy(rowsa1, out_hbm.at[pl.ds(0, chunk)], ws1).wait()

    return k(table, src, dst)


def kernel(x, edge_index, W1a, b1a, W1b, b1b, W2a, b2a, W2b, b2b):
    n_edges = edge_index.shape[1]
    src = edge_index[0].astype(jnp.int32)
    dst = edge_index[1].astype(jnp.int32)

    q = _mm_bias(x, W1a, b1a[None, :], block=1000)
    t = _gather_add(q, src, dst, N_NODES,
                    chunk=104, n_main=3, tail_chunk=16, n_tail=1)
    p32 = _mid_chain(t, W1b, b1b[None, :], W2a, b2a[None, :],
                     block=1000)
    u32 = _gather_add(p32, src, dst, n_edges,
                      chunk=104, n_main=48, tail_chunk=64, n_tail=4)
    return _final_mm(u32, W2b, b2b[:, None], block=6400)


# speedup vs baseline: 1.9444x; 1.0445x over previous
import functools

import jax
import jax.numpy as jnp
from jax import lax
from jax.experimental import pallas as pl
from jax.experimental.pallas import tpu as pltpu
from jax.experimental.pallas import tpu_sc as plsc

N_NODES = 10000
D = 256
EPS = 0.5
NC = 2
NS = 16
NW = NC * NS
MASK_HI = -65536
MASK_LO = 0xFFFF


def _mm_bias(x, W, b, block):
    N, K = x.shape
    M = W.shape[1]
    return pl.pallas_call(
        lambda xr, wr, br, outr: outr.__setitem__(
            ..., jnp.dot(xr[...], wr[...], preferred_element_type=jnp.float32)
            + 0.5 * br[...]),
        grid=(N // block,),
        in_specs=[
            pl.BlockSpec((block, K), lambda i: (i, 0)),
            pl.BlockSpec((K, M), lambda i: (0, 0)),
            pl.BlockSpec((1, M), lambda i: (0, 0)),
        ],
        out_specs=pl.BlockSpec((block, M), lambda i: (i, 0)),
        out_shape=jax.ShapeDtypeStruct((N, M), jnp.float32),
    )(x, W, b)


def _mid_chain(t, W1, b1, W2, b2, block):
    N, K = t.shape
    M = W2.shape[1] // 2

    def rhu16(x):
        return jax.lax.bitcast_convert_type(x, jnp.int32) + 0x8000

    def body(tr, w1r, b1r, w2r, b2r, outr):
        h = jnp.maximum(tr[...], 0.0)
        h = jnp.dot(h, w1r[...], preferred_element_type=jnp.float32) + b1r[...]
        h = jnp.maximum(EPS * h, 0.0)
        w2 = w2r[...]
        b2 = b2r[...]
        pe = (jnp.dot(h, w2[:, :M], preferred_element_type=jnp.float32)
              + 0.5 * b2[:, :M])
        po = (jnp.dot(h, w2[:, M:], preferred_element_type=jnp.float32)
              + 0.5 * b2[:, M:])
        outr[...] = ((rhu16(po) & MASK_HI)
                     | ((rhu16(pe) >> 16) & MASK_LO))

    return pl.pallas_call(
        body,
        grid=(N // block,),
        in_specs=[
            pl.BlockSpec((block, K), lambda i: (i, 0)),
            pl.BlockSpec((K, W1.shape[1]), lambda i: (0, 0)),
            pl.BlockSpec((1, W1.shape[1]), lambda i: (0, 0)),
            pl.BlockSpec((W2.shape[0], 2 * M), lambda i: (0, 0)),
            pl.BlockSpec((1, 2 * M), lambda i: (0, 0)),
        ],
        out_specs=pl.BlockSpec((block, M), lambda i: (i, 0)),
        out_shape=jax.ShapeDtypeStruct((N, M), jnp.int32),
    )(t, W1, b1, W2, b2)


def _final_mm_part(u32, W, b, block, total_n, col_base, prev=None):
    n = u32.shape[0]
    K2, M = W.shape
    K = K2 // 2
    cb = col_base // block

    def body(ur, wr, br, *rest):
        outr = rest[-1]
        a = ur[...]
        lo = jax.lax.bitcast_convert_type(a << 16, jnp.float32)
        hi = jax.lax.bitcast_convert_type(a & MASK_HI, jnp.float32)
        w = wr[...]
        dn = (((0,), (1,)), ((), ()))
        d = (jax.lax.dot_general(w[:K], jnp.maximum(lo, 0.0), dn,
                                 preferred_element_type=jnp.float32)
             + jax.lax.dot_general(w[K:], jnp.maximum(hi, 0.0), dn,
                                   preferred_element_type=jnp.float32))
        outr[...] = EPS * (d + br[...])

    in_specs = [
        pl.BlockSpec((block, K), lambda i: (i, 0)),
        pl.BlockSpec((K2, M), lambda i: (0, 0)),
        pl.BlockSpec((M, 1), lambda i: (0, 0)),
    ]
    args = [u32, W, b]
    aliases = {}
    if prev is not None:
        in_specs.append(pl.BlockSpec(memory_space=pl.ANY))
        args.append(prev)
        aliases = {3: 0}
    return pl.pallas_call(
        body,
        grid=(n // block,),
        in_specs=in_specs,
        out_specs=pl.BlockSpec((M, block), lambda i: (0, i + cb)),
        out_shape=jax.ShapeDtypeStruct((M, total_n), jnp.float32),
        input_output_aliases=aliases,
    )(*args)


def _gather_add(table, src, dst, n_edges, chunk, n_main, tail_chunk, n_tail,
                edge_base=0):
    dt = table.dtype
    width = table.shape[1]
    per_w = n_main * chunk
    tail_base = NW * per_w
    mesh = plsc.VectorSubcoreMesh(core_axis_name="c", subcore_axis_name="s")

    @functools.partial(
        pl.kernel,
        out_type=jax.ShapeDtypeStruct((n_edges, width), dt),
        mesh=mesh,
        scratch_types=[
            pltpu.VMEM((per_w,), jnp.int32),
            pltpu.VMEM((per_w,), jnp.int32),
            pltpu.VMEM((tail_chunk,), jnp.int32),
            pltpu.VMEM((tail_chunk,), jnp.int32),
            pltpu.VMEM((chunk, width), dt),
            pltpu.VMEM((chunk, width), dt),
            pltpu.VMEM((chunk, width), dt),
            pltpu.VMEM((chunk, width), dt),
            pltpu.SemaphoreType.DMA,
            pltpu.SemaphoreType.DMA,
            pltpu.SemaphoreType.DMA,
            pltpu.SemaphoreType.DMA,
            pltpu.SemaphoreType.DMA,
            pltpu.SemaphoreType.DMA,
        ],
    )
    def k(table_hbm, src_hbm, dst_hbm, out_hbm, sidx, didx, tsidx, tdidx,
          rowsa0, rowsa1, rowsb0, rowsb1, ga0, ga1, gb0, gb1, ws0, ws1):
        wid = lax.axis_index("s") * NC + lax.axis_index("c")
        base = wid * per_w
        ebase = edge_base + base
        rowsa = (rowsa0, rowsa1)
        rowsb = (rowsb0, rowsb1)
        ga = (ga0, ga1)
        gb = (gb0, gb1)
        ws = (ws0, ws1)

        def add_rows(ba, bb, n_rows):
            def add_row(r, carry):
                for j in range(width // 16):
                    sl = pl.ds(j * 16, 16)
                    a = ba[r, sl]
                    b_ = bb[r, sl]
                    if dt == jnp.int32:
                        bc = jax.lax.bitcast_convert_type
                        alo = bc(a << 16, jnp.float32)
                        ahi = bc(a & MASK_HI, jnp.float32)
                        blo = bc(b_ << 16, jnp.float32)
                        bhi = bc(b_ & MASK_HI, jnp.float32)
                        rlo = bc(alo + blo, jnp.int32) + 0x8000
                        rhi = bc(ahi + bhi, jnp.int32) + 0x8000
                        ba[r, sl] = ((rhi & MASK_HI)
                                     | ((rlo >> 16) & MASK_LO))
                    else:
                        ba[r, sl] = a + b_
                return carry
            lax.fori_loop(0, n_rows, add_row, 0)

        pltpu.sync_copy(src_hbm.at[pl.ds(ebase, per_w)], sidx)
        pltpu.sync_copy(dst_hbm.at[pl.ds(ebase, per_w)], didx)

        @pl.when(wid < n_tail)
        def _():
            toff = tail_base + wid * tail_chunk
            etoff = edge_base + toff
            pltpu.sync_copy(src_hbm.at[pl.ds(etoff, tail_chunk)], tsidx)
            pltpu.sync_copy(dst_hbm.at[pl.ds(etoff, tail_chunk)], tdidx)
            tra = rowsa0.at[pl.ds(0, tail_chunk)]
            trb = rowsb0.at[pl.ds(0, tail_chunk)]
            pltpu.async_copy(table_hbm.at[tsidx], tra, ga0)
            pltpu.async_copy(table_hbm.at[tdidx], trb, gb0)
            pltpu.make_async_copy(
                table_hbm.at[pl.ds(0, tail_chunk)], tra, ga0).wait()
            pltpu.make_async_copy(
                table_hbm.at[pl.ds(0, tail_chunk)], trb, gb0).wait()
            add_rows(rowsa0, rowsb0, tail_chunk)
            pltpu.sync_copy(tra, out_hbm.at[pl.ds(toff, tail_chunk)])

        pltpu.async_copy(table_hbm.at[sidx.at[pl.ds(0, chunk)]], rowsa0, ga0)
        pltpu.async_copy(table_hbm.at[didx.at[pl.ds(0, chunk)]], rowsb0, gb0)

        def step(c, b, nb):
            pltpu.make_async_copy(
                table_hbm.at[pl.ds(0, chunk)], rowsa[b], ga[b]).wait()
            pltpu.make_async_copy(
                table_hbm.at[pl.ds(0, chunk)], rowsb[b], gb[b]).wait()
            @pl.when(c + 1 < n_main)
            def _():
                @pl.when(c >= 1)
                def _():
                    pltpu.make_async_copy(
                        rowsa[nb], out_hbm.at[pl.ds(0, chunk)], ws[nb]).wait()
                pltpu.async_copy(
                    table_hbm.at[sidx.at[pl.ds((c + 1) * chunk, chunk)]],
                    rowsa[nb], ga[nb])
                pltpu.async_copy(
                    table_hbm.at[didx.at[pl.ds((c + 1) * chunk, chunk)]],
                    rowsb[nb], gb[nb])
            add_rows(rowsa[b], rowsb[b], chunk)
            pltpu.async_copy(
                rowsa[b], out_hbm.at[pl.ds(base + c * chunk, chunk)], ws[b])

        def pair(c2, carry):
            c = c2 * 2
            step(c, 0, 1)
            @pl.when(c + 1 < n_main)
            def _():
                step(c + 1, 1, 0)
            return carry

        lax.fori_loop(0, (n_main + 1) // 2, pair, 0)

        pltpu.make_async_copy(rowsa0, out_hbm.at[pl.ds(0, chunk)], ws0).wait()
        pltpu.make_async_copy(rowsa1, out_hbm.at[pl.ds(0, chunk)], ws1).wait()

    return k(table, src, dst)


def kernel(x, edge_index, W1a, b1a, W1b, b1b, W2a, b2a, W2b, b2b):
    n_edges = edge_index.shape[1]
    src = edge_index[0].astype(jnp.int32)
    dst = edge_index[1].astype(jnp.int32)

    q = _mm_bias(x, W1a, b1a[None, :], block=1000)
    t = _gather_add(q, src, dst, N_NODES,
                    chunk=104, n_main=3, tail_chunk=16, n_tail=1)
    p32 = _mid_chain(t, W1b, b1b[None, :], W2a, b2a[None, :],
                     block=1000)
    h1, h2 = 76800, 83200
    u1 = _gather_add(p32, src, dst, h1,
                     chunk=96, n_main=25, tail_chunk=8, n_tail=0)
    u2 = _gather_add(p32, src, dst, h2,
                     chunk=104, n_main=25, tail_chunk=8, n_tail=0,
                     edge_base=h1)
    o1 = _final_mm_part(u1, W2b, b2b[:, None], 6400, n_edges, 0)
    o2 = _final_mm_part(u2, W2b, b2b[:, None], 6400, n_edges, h1, prev=o1)
    return o2.T


# speedup vs baseline: 1.9684x; 1.0123x over previous
import functools

import jax
import jax.numpy as jnp
from jax import lax
from jax.experimental import pallas as pl
from jax.experimental.pallas import tpu as pltpu
from jax.experimental.pallas import tpu_sc as plsc

N_NODES = 10000
D = 256
EPS = 0.5
NC = 2
NS = 16
NW = NC * NS
MASK_HI = -65536
MASK_LO = 0xFFFF


def _mm_bias(x, W, b, block):
    N, K = x.shape
    M = W.shape[1]
    return pl.pallas_call(
        lambda xr, wr, br, outr: outr.__setitem__(
            ..., jnp.dot(xr[...], wr[...], preferred_element_type=jnp.float32)
            + 0.5 * br[...]),
        grid=(N // block,),
        in_specs=[
            pl.BlockSpec((block, K), lambda i: (i, 0)),
            pl.BlockSpec((K, M), lambda i: (0, 0)),
            pl.BlockSpec((1, M), lambda i: (0, 0)),
        ],
        out_specs=pl.BlockSpec((block, M), lambda i: (i, 0)),
        out_shape=jax.ShapeDtypeStruct((N, M), jnp.float32),
    )(x, W, b)


def _mid_chain(t, W1, b1, W2, b2, block):
    N, K = t.shape
    M = W2.shape[1] // 2

    def rhu16(x):
        return jax.lax.bitcast_convert_type(x, jnp.int32) + 0x8000

    def body(tr, w1r, b1r, w2r, b2r, outr):
        h = jnp.maximum(tr[...], 0.0)
        h = jnp.dot(h, w1r[...], preferred_element_type=jnp.float32) + b1r[...]
        h = jnp.maximum(EPS * h, 0.0)
        w2 = w2r[...]
        b2 = b2r[...]
        pe = (jnp.dot(h, w2[:, :M], preferred_element_type=jnp.float32)
              + 0.5 * b2[:, :M])
        po = (jnp.dot(h, w2[:, M:], preferred_element_type=jnp.float32)
              + 0.5 * b2[:, M:])
        outr[...] = ((rhu16(po) & MASK_HI)
                     | ((rhu16(pe) >> 16) & MASK_LO))

    return pl.pallas_call(
        body,
        grid=(N // block,),
        in_specs=[
            pl.BlockSpec((block, K), lambda i: (i, 0)),
            pl.BlockSpec((K, W1.shape[1]), lambda i: (0, 0)),
            pl.BlockSpec((1, W1.shape[1]), lambda i: (0, 0)),
            pl.BlockSpec((W2.shape[0], 2 * M), lambda i: (0, 0)),
            pl.BlockSpec((1, 2 * M), lambda i: (0, 0)),
        ],
        out_specs=pl.BlockSpec((block, M), lambda i: (i, 0)),
        out_shape=jax.ShapeDtypeStruct((N, M), jnp.int32),
    )(t, W1, b1, W2, b2)


def _final_mm_part(u32, W, b, block, total_n, col_base, prev=None):
    n = u32.shape[0]
    K2, M = W.shape
    K = K2 // 2
    cb = col_base // block

    def body(ur, wr, br, *rest):
        outr = rest[-1]
        a = ur[...]
        lo = jax.lax.bitcast_convert_type(a << 16, jnp.float32)
        hi = jax.lax.bitcast_convert_type(a & MASK_HI, jnp.float32)
        w = wr[...]
        dn = (((0,), (1,)), ((), ()))
        d = (jax.lax.dot_general(w[:K], jnp.maximum(lo, 0.0), dn,
                                 preferred_element_type=jnp.float32)
             + jax.lax.dot_general(w[K:], jnp.maximum(hi, 0.0), dn,
                                   preferred_element_type=jnp.float32))
        outr[...] = EPS * (d + br[...])

    in_specs = [
        pl.BlockSpec((block, K), lambda i: (i, 0)),
        pl.BlockSpec((K2, M), lambda i: (0, 0)),
        pl.BlockSpec((M, 1), lambda i: (0, 0)),
    ]
    args = [u32, W, b]
    aliases = {}
    if prev is not None:
        in_specs.append(pl.BlockSpec(memory_space=pl.ANY))
        args.append(prev)
        aliases = {3: 0}
    return pl.pallas_call(
        body,
        grid=(n // block,),
        in_specs=in_specs,
        out_specs=pl.BlockSpec((M, block), lambda i: (0, i + cb)),
        out_shape=jax.ShapeDtypeStruct((M, total_n), jnp.float32),
        input_output_aliases=aliases,
    )(*args)


def _gather_add(table, src, dst, n_edges, chunk, n_main, tail_chunk, n_tail,
                edge_base=0):
    dt = table.dtype
    width = table.shape[1]
    per_w = n_main * chunk
    tail_base = NW * per_w
    mesh = plsc.VectorSubcoreMesh(core_axis_name="c", subcore_axis_name="s")

    @functools.partial(
        pl.kernel,
        out_type=jax.ShapeDtypeStruct((n_edges, width), dt),
        mesh=mesh,
        scratch_types=[
            pltpu.VMEM((per_w,), jnp.int32),
            pltpu.VMEM((per_w,), jnp.int32),
            pltpu.VMEM((tail_chunk,), jnp.int32),
            pltpu.VMEM((tail_chunk,), jnp.int32),
            pltpu.VMEM((chunk, width), dt),
            pltpu.VMEM((chunk, width), dt),
            pltpu.VMEM((chunk, width), dt),
            pltpu.VMEM((chunk, width), dt),
            pltpu.SemaphoreType.DMA,
            pltpu.SemaphoreType.DMA,
            pltpu.SemaphoreType.DMA,
            pltpu.SemaphoreType.DMA,
            pltpu.SemaphoreType.DMA,
            pltpu.SemaphoreType.DMA,
        ],
    )
    def k(table_hbm, src_hbm, dst_hbm, out_hbm, sidx, didx, tsidx, tdidx,
          rowsa0, rowsa1, rowsb0, rowsb1, ga0, ga1, gb0, gb1, ws0, ws1):
        wid = lax.axis_index("s") * NC + lax.axis_index("c")
        base = wid * per_w
        ebase = edge_base + base
        rowsa = (rowsa0, rowsa1)
        rowsb = (rowsb0, rowsb1)
        ga = (ga0, ga1)
        gb = (gb0, gb1)
        ws = (ws0, ws1)

        def add_rows(ba, bb, n_rows):
            def add_row(r, carry):
                for j in range(width // 16):
                    sl = pl.ds(j * 16, 16)
                    a = ba[r, sl]
                    b_ = bb[r, sl]
                    if dt == jnp.int32:
                        bc = jax.lax.bitcast_convert_type
                        alo = bc(a << 16, jnp.float32)
                        ahi = bc(a & MASK_HI, jnp.float32)
                        blo = bc(b_ << 16, jnp.float32)
                        bhi = bc(b_ & MASK_HI, jnp.float32)
                        rlo = bc(alo + blo, jnp.int32) + 0x8000
                        rhi = bc(ahi + bhi, jnp.int32) + 0x8000
                        ba[r, sl] = ((rhi & MASK_HI)
                                     | ((rlo >> 16) & MASK_LO))
                    else:
                        ba[r, sl] = a + b_
                return carry
            lax.fori_loop(0, n_rows, add_row, 0)

        pltpu.sync_copy(src_hbm.at[pl.ds(ebase, per_w)], sidx)
        pltpu.sync_copy(dst_hbm.at[pl.ds(ebase, per_w)], didx)

        @pl.when(wid < n_tail)
        def _():
            toff = tail_base + wid * tail_chunk
            etoff = edge_base + toff
            pltpu.sync_copy(src_hbm.at[pl.ds(etoff, tail_chunk)], tsidx)
            pltpu.sync_copy(dst_hbm.at[pl.ds(etoff, tail_chunk)], tdidx)
            tra = rowsa0.at[pl.ds(0, tail_chunk)]
            trb = rowsb0.at[pl.ds(0, tail_chunk)]
            pltpu.async_copy(table_hbm.at[tsidx], tra, ga0)
            pltpu.async_copy(table_hbm.at[tdidx], trb, gb0)
            pltpu.make_async_copy(
                table_hbm.at[pl.ds(0, tail_chunk)], tra, ga0).wait()
            pltpu.make_async_copy(
                table_hbm.at[pl.ds(0, tail_chunk)], trb, gb0).wait()
            add_rows(rowsa0, rowsb0, tail_chunk)
            pltpu.sync_copy(tra, out_hbm.at[pl.ds(toff, tail_chunk)])

        pltpu.async_copy(table_hbm.at[sidx.at[pl.ds(0, chunk)]], rowsa0, ga0)
        pltpu.async_copy(table_hbm.at[didx.at[pl.ds(0, chunk)]], rowsb0, gb0)

        def step(c, b, nb):
            pltpu.make_async_copy(
                table_hbm.at[pl.ds(0, chunk)], rowsa[b], ga[b]).wait()
            pltpu.make_async_copy(
                table_hbm.at[pl.ds(0, chunk)], rowsb[b], gb[b]).wait()
            @pl.when(c + 1 < n_main)
            def _():
                @pl.when(c >= 1)
                def _():
                    pltpu.make_async_copy(
                        rowsa[nb], out_hbm.at[pl.ds(0, chunk)], ws[nb]).wait()
                pltpu.async_copy(
                    table_hbm.at[sidx.at[pl.ds((c + 1) * chunk, chunk)]],
                    rowsa[nb], ga[nb])
                pltpu.async_copy(
                    table_hbm.at[didx.at[pl.ds((c + 1) * chunk, chunk)]],
                    rowsb[nb], gb[nb])
            add_rows(rowsa[b], rowsb[b], chunk)
            pltpu.async_copy(
                rowsa[b], out_hbm.at[pl.ds(base + c * chunk, chunk)], ws[b])

        def pair(c2, carry):
            c = c2 * 2
            step(c, 0, 1)
            @pl.when(c + 1 < n_main)
            def _():
                step(c + 1, 1, 0)
            return carry

        lax.fori_loop(0, (n_main + 1) // 2, pair, 0)

        pltpu.make_async_copy(rowsa0, out_hbm.at[pl.ds(0, chunk)], ws0).wait()
        pltpu.make_async_copy(rowsa1, out_hbm.at[pl.ds(0, chunk)], ws1).wait()

    return k(table, src, dst)


def kernel(x, edge_index, W1a, b1a, W1b, b1b, W2a, b2a, W2b, b2b):
    n_edges = edge_index.shape[1]
    src = edge_index[0].astype(jnp.int32)
    dst = edge_index[1].astype(jnp.int32)

    q = _mm_bias(x, W1a, b1a[None, :], block=1000)
    t = _gather_add(q, src, dst, N_NODES,
                    chunk=104, n_main=3, tail_chunk=16, n_tail=1)
    p32 = _mid_chain(t, W1b, b1b[None, :], W2a, b2a[None, :],
                     block=1000)
    h1, h2 = 102400, 57600
    u1 = _gather_add(p32, src, dst, h1,
                     chunk=128, n_main=25, tail_chunk=8, n_tail=0)
    u2 = _gather_add(p32, src, dst, h2,
                     chunk=72, n_main=25, tail_chunk=8, n_tail=0,
                     edge_base=h1)
    o1 = _final_mm_part(u1, W2b, b2b[:, None], 6400, n_edges, 0)
    o2 = _final_mm_part(u2, W2b, b2b[:, None], 6400, n_edges, h1, prev=o1)
    return o2.T


# speedup vs baseline: 2.0755x; 1.0544x over previous
import functools

import jax
import jax.numpy as jnp
from jax import lax
from jax.experimental import pallas as pl
from jax.experimental.pallas import tpu as pltpu
from jax.experimental.pallas import tpu_sc as plsc

N_NODES = 10000
D = 256
EPS = 0.5
NC = 2
NS = 16
NW = NC * NS
MASK_HI = -65536
MASK_LO = 0xFFFF


def _mm_bias(x, W, b, edges):
    N, K = x.shape
    M = W.shape[1]
    E = edges.shape[1]

    def body(xr, wr, br, er, qr, sr, dr):
        qr[...] = (jnp.dot(xr[...], wr[...], preferred_element_type=jnp.float32)
                   + 0.5 * br[...])
        sr[...] = er[0, :]
        dr[...] = er[1, :]

    return pl.pallas_call(
        body,
        out_shape=[
            jax.ShapeDtypeStruct((N, M), jnp.float32),
            jax.ShapeDtypeStruct((E,), jnp.int32),
            jax.ShapeDtypeStruct((E,), jnp.int32),
        ],
    )(x, W, b, edges)


def _mid_chain(t, W1, b1, W2, b2, block):
    N, K = t.shape
    M = W2.shape[1] // 2

    def rhu16(x):
        return jax.lax.bitcast_convert_type(x, jnp.int32) + 0x8000

    def body(tr, w1r, b1r, w2r, b2r, outr):
        h = jnp.maximum(tr[...], 0.0)
        h = jnp.dot(h, w1r[...], preferred_element_type=jnp.float32) + b1r[...]
        h = jnp.maximum(EPS * h, 0.0)
        w2 = w2r[...]
        b2 = b2r[...]
        pe = (jnp.dot(h, w2[:, :M], preferred_element_type=jnp.float32)
              + 0.5 * b2[:, :M])
        po = (jnp.dot(h, w2[:, M:], preferred_element_type=jnp.float32)
              + 0.5 * b2[:, M:])
        outr[...] = ((rhu16(po) & MASK_HI)
                     | ((rhu16(pe) >> 16) & MASK_LO))

    return pl.pallas_call(
        body,
        grid=(N // block,),
        in_specs=[
            pl.BlockSpec((block, K), lambda i: (i, 0)),
            pl.BlockSpec((K, W1.shape[1]), lambda i: (0, 0)),
            pl.BlockSpec((1, W1.shape[1]), lambda i: (0, 0)),
            pl.BlockSpec((W2.shape[0], 2 * M), lambda i: (0, 0)),
            pl.BlockSpec((1, 2 * M), lambda i: (0, 0)),
        ],
        out_specs=pl.BlockSpec((block, M), lambda i: (i, 0)),
        out_shape=jax.ShapeDtypeStruct((N, M), jnp.int32),
    )(t, W1, b1, W2, b2)


def _final_mm_part(u32, W, b, block, total_n, col_base, prev=None):
    n = u32.shape[0]
    K2, M = W.shape
    K = K2 // 2
    cb = col_base // block

    def body(ur, wr, br, *rest):
        outr = rest[-1]
        a = ur[...]
        lo = jax.lax.bitcast_convert_type(a << 16, jnp.float32)
        hi = jax.lax.bitcast_convert_type(a & MASK_HI, jnp.float32)
        w = wr[...]
        dn = (((0,), (1,)), ((), ()))
        d = (jax.lax.dot_general(w[:K], jnp.maximum(lo, 0.0), dn,
                                 preferred_element_type=jnp.float32)
             + jax.lax.dot_general(w[K:], jnp.maximum(hi, 0.0), dn,
                                   preferred_element_type=jnp.float32))
        outr[...] = EPS * (d + br[...])

    in_specs = [
        pl.BlockSpec((block, K), lambda i: (i, 0)),
        pl.BlockSpec((K2, M), lambda i: (0, 0)),
        pl.BlockSpec((M, 1), lambda i: (0, 0)),
    ]
    args = [u32, W, b]
    aliases = {}
    if prev is not None:
        in_specs.append(pl.BlockSpec(memory_space=pl.ANY))
        args.append(prev)
        aliases = {3: 0}
    return pl.pallas_call(
        body,
        grid=(n // block,),
        in_specs=in_specs,
        out_specs=pl.BlockSpec((M, block), lambda i: (0, i + cb)),
        out_shape=jax.ShapeDtypeStruct((M, total_n), jnp.float32),
        input_output_aliases=aliases,
    )(*args)


def _gather_add(table, src, dst, n_edges, chunk, n_main, tail_chunk, n_tail,
                edge_base=0):
    dt = table.dtype
    width = table.shape[1]
    per_w = n_main * chunk
    tail_base = NW * per_w
    mesh = plsc.VectorSubcoreMesh(core_axis_name="c", subcore_axis_name="s")

    @functools.partial(
        pl.kernel,
        out_type=jax.ShapeDtypeStruct((n_edges, width), dt),
        mesh=mesh,
        scratch_types=[
            pltpu.VMEM((per_w,), jnp.int32),
            pltpu.VMEM((per_w,), jnp.int32),
            pltpu.VMEM((tail_chunk,), jnp.int32),
            pltpu.VMEM((tail_chunk,), jnp.int32),
            pltpu.VMEM((chunk, width), dt),
            pltpu.VMEM((chunk, width), dt),
            pltpu.VMEM((chunk, width), dt),
            pltpu.VMEM((chunk, width), dt),
            pltpu.SemaphoreType.DMA,
            pltpu.SemaphoreType.DMA,
            pltpu.SemaphoreType.DMA,
            pltpu.SemaphoreType.DMA,
            pltpu.SemaphoreType.DMA,
            pltpu.SemaphoreType.DMA,
        ],
    )
    def k(table_hbm, src_hbm, dst_hbm, out_hbm, sidx, didx, tsidx, tdidx,
          rowsa0, rowsa1, rowsb0, rowsb1, ga0, ga1, gb0, gb1, ws0, ws1):
        wid = lax.axis_index("s") * NC + lax.axis_index("c")
        base = wid * per_w
        ebase = edge_base + base
        rowsa = (rowsa0, rowsa1)
        rowsb = (rowsb0, rowsb1)
        ga = (ga0, ga1)
        gb = (gb0, gb1)
        ws = (ws0, ws1)

        def add_rows(ba, bb, n_rows):
            def add_row(r, carry):
                for j in range(width // 16):
                    sl = pl.ds(j * 16, 16)
                    a = ba[r, sl]
                    b_ = bb[r, sl]
                    if dt == jnp.int32:
                        bc = jax.lax.bitcast_convert_type
                        alo = bc(a << 16, jnp.float32)
                        ahi = bc(a & MASK_HI, jnp.float32)
                        blo = bc(b_ << 16, jnp.float32)
                        bhi = bc(b_ & MASK_HI, jnp.float32)
                        rlo = bc(alo + blo, jnp.int32) + 0x8000
                        rhi = bc(ahi + bhi, jnp.int32) + 0x8000
                        ba[r, sl] = ((rhi & MASK_HI)
                                     | ((rlo >> 16) & MASK_LO))
                    else:
                        ba[r, sl] = a + b_
                return carry
            lax.fori_loop(0, n_rows, add_row, 0)

        pltpu.sync_copy(src_hbm.at[pl.ds(ebase, per_w)], sidx)
        pltpu.sync_copy(dst_hbm.at[pl.ds(ebase, per_w)], didx)

        @pl.when(wid < n_tail)
        def _():
            toff = tail_base + wid * tail_chunk
            etoff = edge_base + toff
            pltpu.sync_copy(src_hbm.at[pl.ds(etoff, tail_chunk)], tsidx)
            pltpu.sync_copy(dst_hbm.at[pl.ds(etoff, tail_chunk)], tdidx)
            tra = rowsa0.at[pl.ds(0, tail_chunk)]
            trb = rowsb0.at[pl.ds(0, tail_chunk)]
            pltpu.async_copy(table_hbm.at[tsidx], tra, ga0)
            pltpu.async_copy(table_hbm.at[tdidx], trb, gb0)
            pltpu.make_async_copy(
                table_hbm.at[pl.ds(0, tail_chunk)], tra, ga0).wait()
            pltpu.make_async_copy(
                table_hbm.at[pl.ds(0, tail_chunk)], trb, gb0).wait()
            add_rows(rowsa0, rowsb0, tail_chunk)
            pltpu.sync_copy(tra, out_hbm.at[pl.ds(toff, tail_chunk)])

        pltpu.async_copy(table_hbm.at[sidx.at[pl.ds(0, chunk)]], rowsa0, ga0)
        pltpu.async_copy(table_hbm.at[didx.at[pl.ds(0, chunk)]], rowsb0, gb0)

        def step(c, b, nb):
            pltpu.make_async_copy(
                table_hbm.at[pl.ds(0, chunk)], rowsa[b], ga[b]).wait()
            pltpu.make_async_copy(
                table_hbm.at[pl.ds(0, chunk)], rowsb[b], gb[b]).wait()
            @pl.when(c + 1 < n_main)
            def _():
                @pl.when(c >= 1)
                def _():
                    pltpu.make_async_copy(
                        rowsa[nb], out_hbm.at[pl.ds(0, chunk)], ws[nb]).wait()
                pltpu.async_copy(
                    table_hbm.at[sidx.at[pl.ds((c + 1) * chunk, chunk)]],
                    rowsa[nb], ga[nb])
                pltpu.async_copy(
                    table_hbm.at[didx.at[pl.ds((c + 1) * chunk, chunk)]],
                    rowsb[nb], gb[nb])
            add_rows(rowsa[b], rowsb[b], chunk)
            pltpu.async_copy(
                rowsa[b], out_hbm.at[pl.ds(base + c * chunk, chunk)], ws[b])

        def pair(c2, carry):
            c = c2 * 2
            step(c, 0, 1)
            @pl.when(c + 1 < n_main)
            def _():
                step(c + 1, 1, 0)
            return carry

        lax.fori_loop(0, (n_main + 1) // 2, pair, 0)

        pltpu.make_async_copy(rowsa0, out_hbm.at[pl.ds(0, chunk)], ws0).wait()
        pltpu.make_async_copy(rowsa1, out_hbm.at[pl.ds(0, chunk)], ws1).wait()

    return k(table, src, dst)


def kernel(x, edge_index, W1a, b1a, W1b, b1b, W2a, b2a, W2b, b2b):
    n_edges = edge_index.shape[1]
    q, src, dst = _mm_bias(x, W1a, b1a[None, :],
                           edge_index.astype(jnp.int32))
    t = _gather_add(q, src, dst, N_NODES,
                    chunk=104, n_main=3, tail_chunk=16, n_tail=1)
    p32 = _mid_chain(t, W1b, b1b[None, :], W2a, b2a[None, :],
                     block=1000)
    h1, h2 = 102400, 57600
    u1 = _gather_add(p32, src, dst, h1,
                     chunk=128, n_main=25, tail_chunk=8, n_tail=0)
    u2 = _gather_add(p32, src, dst, h2,
                     chunk=72, n_main=25, tail_chunk=8, n_tail=0,
                     edge_base=h1)
    o1 = _final_mm_part(u1, W2b, b2b[:, None], 6400, n_edges, 0)
    o2 = _final_mm_part(u2, W2b, b2b[:, None], 6400, n_edges, h1, prev=o1)
    return o2.T


# speedup vs baseline: 2.1878x; 1.0541x over previous
import functools

import jax
import jax.numpy as jnp
from jax import lax
from jax.experimental import pallas as pl
from jax.experimental.pallas import tpu as pltpu
from jax.experimental.pallas import tpu_sc as plsc

N_NODES = 10000
D = 256
EPS = 0.5
NC = 2
NS = 16
NW = NC * NS
MASK_HI = -65536
MASK_LO = 0xFFFF


def _mm_bias(x, W, b, edges):
    N, K = x.shape
    M = W.shape[1]
    E = edges.shape[1]

    def body(xr, wr, br, er, qr, sr, dr):
        qr[...] = (jnp.dot(xr[...], wr[...], preferred_element_type=jnp.float32)
                   + 0.5 * br[...])
        sr[...] = er[0, :]
        dr[...] = er[1, :]

    return pl.pallas_call(
        body,
        out_shape=[
            jax.ShapeDtypeStruct((N, M), jnp.float32),
            jax.ShapeDtypeStruct((E,), jnp.int32),
            jax.ShapeDtypeStruct((E,), jnp.int32),
        ],
    )(x, W, b, edges)


def _mid_chain(t, W1, b1, W2, b2, block):
    N, K = t.shape
    M = W2.shape[1] // 2

    def rhu16(x):
        return jax.lax.bitcast_convert_type(x, jnp.int32) + 0x8000

    def body(tr, w1r, b1r, w2r, b2r, outr):
        h = jnp.maximum(tr[...], 0.0)
        h = jnp.dot(h, w1r[...], preferred_element_type=jnp.float32) + b1r[...]
        h = jnp.maximum(EPS * h, 0.0)
        w2 = w2r[...]
        b2 = b2r[...]
        pe = (jnp.dot(h, w2[:, :M], preferred_element_type=jnp.float32)
              + 0.5 * b2[:, :M])
        po = (jnp.dot(h, w2[:, M:], preferred_element_type=jnp.float32)
              + 0.5 * b2[:, M:])
        outr[...] = ((rhu16(po) & MASK_HI)
                     | ((rhu16(pe) >> 16) & MASK_LO))

    return pl.pallas_call(
        body,
        grid=(N // block,),
        in_specs=[
            pl.BlockSpec((block, K), lambda i: (i, 0)),
            pl.BlockSpec((K, W1.shape[1]), lambda i: (0, 0)),
            pl.BlockSpec((1, W1.shape[1]), lambda i: (0, 0)),
            pl.BlockSpec((W2.shape[0], 2 * M), lambda i: (0, 0)),
            pl.BlockSpec((1, 2 * M), lambda i: (0, 0)),
        ],
        out_specs=pl.BlockSpec((block, M), lambda i: (i, 0)),
        out_shape=jax.ShapeDtypeStruct((N, M), jnp.int32),
    )(t, W1, b1, W2, b2)


def _final_mm_part(u32, W, b, block, total_n, col_base, prev=None):
    n = u32.shape[0]
    K2, M = W.shape
    K = K2 // 2
    cb = col_base // block

    def body(ur, wr, br, *rest):
        outr = rest[-1]
        a = ur[...]
        lo = jax.lax.bitcast_convert_type(a << 16, jnp.float32)
        hi = jax.lax.bitcast_convert_type(a & MASK_HI, jnp.float32)
        w = wr[...]
        dn = (((0,), (1,)), ((), ()))
        d = (jax.lax.dot_general(w[:K], jnp.maximum(lo, 0.0), dn,
                                 preferred_element_type=jnp.float32)
             + jax.lax.dot_general(w[K:], jnp.maximum(hi, 0.0), dn,
                                   preferred_element_type=jnp.float32))
        outr[...] = EPS * (d + br[...])

    in_specs = [
        pl.BlockSpec((block, K), lambda i: (i, 0)),
        pl.BlockSpec((K2, M), lambda i: (0, 0)),
        pl.BlockSpec((M, 1), lambda i: (0, 0)),
    ]
    args = [u32, W, b]
    aliases = {}
    if prev is not None:
        in_specs.append(pl.BlockSpec(memory_space=pl.ANY))
        args.append(prev)
        aliases = {3: 0}
    return pl.pallas_call(
        body,
        grid=(n // block,),
        in_specs=in_specs,
        out_specs=pl.BlockSpec((M, block), lambda i: (0, i + cb)),
        out_shape=jax.ShapeDtypeStruct((M, total_n), jnp.float32),
        input_output_aliases=aliases,
    )(*args)


def _gather_add(table, src, dst, n_edges, chunk, n_main, tail_chunk, n_tail,
                edge_base=0):
    dt = table.dtype
    width = table.shape[1]
    per_w = n_main * chunk
    tail_base = NW * per_w
    mesh = plsc.VectorSubcoreMesh(core_axis_name="c", subcore_axis_name="s")

    @functools.partial(
        pl.kernel,
        out_type=jax.ShapeDtypeStruct((n_edges, width), dt),
        mesh=mesh,
        scratch_types=[
            pltpu.VMEM((per_w,), jnp.int32),
            pltpu.VMEM((per_w,), jnp.int32),
            pltpu.VMEM((tail_chunk,), jnp.int32),
            pltpu.VMEM((tail_chunk,), jnp.int32),
            pltpu.VMEM((chunk, width), dt),
            pltpu.VMEM((chunk, width), dt),
            pltpu.VMEM((chunk, width), dt),
            pltpu.VMEM((chunk, width), dt),
            pltpu.SemaphoreType.DMA,
            pltpu.SemaphoreType.DMA,
            pltpu.SemaphoreType.DMA,
            pltpu.SemaphoreType.DMA,
            pltpu.SemaphoreType.DMA,
            pltpu.SemaphoreType.DMA,
        ],
    )
    def k(table_hbm, src_hbm, dst_hbm, out_hbm, sidx, didx, tsidx, tdidx,
          rowsa0, rowsa1, rowsb0, rowsb1, ga0, ga1, gb0, gb1, ws0, ws1):
        wid = lax.axis_index("s") * NC + lax.axis_index("c")
        base = wid * per_w
        ebase = edge_base + base
        rowsa = (rowsa0, rowsa1)
        rowsb = (rowsb0, rowsb1)
        ga = (ga0, ga1)
        gb = (gb0, gb1)
        ws = (ws0, ws1)

        def add_rows(ba, bb, n_rows):
            def add_row(r, carry):
                for j in range(width // 16):
                    sl = pl.ds(j * 16, 16)
                    a = ba[r, sl]
                    b_ = bb[r, sl]
                    if dt == jnp.int32:
                        bc = jax.lax.bitcast_convert_type
                        alo = bc(a << 16, jnp.float32)
                        ahi = bc(a & MASK_HI, jnp.float32)
                        blo = bc(b_ << 16, jnp.float32)
                        bhi = bc(b_ & MASK_HI, jnp.float32)
                        rlo = bc(alo + blo, jnp.int32)
                        rhi = bc(ahi + bhi, jnp.int32)
                        ba[r, sl] = ((rhi & MASK_HI)
                                     | ((rlo >> 16) & MASK_LO))
                    else:
                        ba[r, sl] = a + b_
                return carry
            lax.fori_loop(0, n_rows, add_row, 0)

        pltpu.sync_copy(src_hbm.at[pl.ds(ebase, per_w)], sidx)
        pltpu.sync_copy(dst_hbm.at[pl.ds(ebase, per_w)], didx)

        @pl.when(wid < n_tail)
        def _():
            toff = tail_base + wid * tail_chunk
            etoff = edge_base + toff
            pltpu.sync_copy(src_hbm.at[pl.ds(etoff, tail_chunk)], tsidx)
            pltpu.sync_copy(dst_hbm.at[pl.ds(etoff, tail_chunk)], tdidx)
            tra = rowsa0.at[pl.ds(0, tail_chunk)]
            trb = rowsb0.at[pl.ds(0, tail_chunk)]
            pltpu.async_copy(table_hbm.at[tsidx], tra, ga0)
            pltpu.async_copy(table_hbm.at[tdidx], trb, gb0)
            pltpu.make_async_copy(
                table_hbm.at[pl.ds(0, tail_chunk)], tra, ga0).wait()
            pltpu.make_async_copy(
                table_hbm.at[pl.ds(0, tail_chunk)], trb, gb0).wait()
            add_rows(rowsa0, rowsb0, tail_chunk)
            pltpu.sync_copy(tra, out_hbm.at[pl.ds(toff, tail_chunk)])

        pltpu.async_copy(table_hbm.at[sidx.at[pl.ds(0, chunk)]], rowsa0, ga0)
        pltpu.async_copy(table_hbm.at[didx.at[pl.ds(0, chunk)]], rowsb0, gb0)

        def step(c, b, nb):
            pltpu.make_async_copy(
                table_hbm.at[pl.ds(0, chunk)], rowsa[b], ga[b]).wait()
            pltpu.make_async_copy(
                table_hbm.at[pl.ds(0, chunk)], rowsb[b], gb[b]).wait()
            @pl.when(c + 1 < n_main)
            def _():
                @pl.when(c >= 1)
                def _():
                    pltpu.make_async_copy(
                        rowsa[nb], out_hbm.at[pl.ds(0, chunk)], ws[nb]).wait()
                pltpu.async_copy(
                    table_hbm.at[sidx.at[pl.ds((c + 1) * chunk, chunk)]],
                    rowsa[nb], ga[nb])
                pltpu.async_copy(
                    table_hbm.at[didx.at[pl.ds((c + 1) * chunk, chunk)]],
                    rowsb[nb], gb[nb])
            add_rows(rowsa[b], rowsb[b], chunk)
            pltpu.async_copy(
                rowsa[b], out_hbm.at[pl.ds(base + c * chunk, chunk)], ws[b])

        def pair(c2, carry):
            c = c2 * 2
            step(c, 0, 1)
            @pl.when(c + 1 < n_main)
            def _():
                step(c + 1, 1, 0)
            return carry

        lax.fori_loop(0, (n_main + 1) // 2, pair, 0)

        pltpu.make_async_copy(rowsa0, out_hbm.at[pl.ds(0, chunk)], ws0).wait()
        pltpu.make_async_copy(rowsa1, out_hbm.at[pl.ds(0, chunk)], ws1).wait()

    return k(table, src, dst)


def kernel(x, edge_index, W1a, b1a, W1b, b1b, W2a, b2a, W2b, b2b):
    n_edges = edge_index.shape[1]
    q, src, dst = _mm_bias(x, W1a, b1a[None, :],
                           edge_index.astype(jnp.int32))
    t = _gather_add(q, src, dst, N_NODES,
                    chunk=104, n_main=3, tail_chunk=16, n_tail=1)
    p32 = _mid_chain(t, W1b, b1b[None, :], W2a, b2a[None, :],
                     block=1000)
    h1, h2 = 102400, 57600
    u1 = _gather_add(p32, src, dst, h1,
                     chunk=128, n_main=25, tail_chunk=8, n_tail=0)
    u2 = _gather_add(p32, src, dst, h2,
                     chunk=72, n_main=25, tail_chunk=8, n_tail=0,
                     edge_base=h1)
    o1 = _final_mm_part(u1, W2b, b2b[:, None], 6400, n_edges, 0)
    o2 = _final_mm_part(u2, W2b, b2b[:, None], 6400, n_edges, h1, prev=o1)
    return o2.T
